# baseline (speedup 1.0000x reference)
import jax
import jax.numpy as jnp
from jax.experimental import pallas as pl

_B, _N, _C, _K = 2, 1024, 64, 16


def _ptl_kernel(xp_ref, post_ref, wt_ref, bt_ref, wqg_ref, cqg_ref,
                pw1t_ref, pb1_ref, wpg_ref, gw2t_ref, pw2t_ref, pb2_ref,
                out_ref):
    iota = jax.lax.broadcasted_iota(jnp.int32, (_N, _N), 1).astype(jnp.float32)
    big = jnp.float32(_N)
    inf = jnp.float32(jnp.inf)

    dcur = []
    tbl = []
    qpe = []
    qg = []
    for b in range(_B):
        XP = xp_ref[b]
        xb = XP[:, 0:_C]
        posb = XP[:, _C:_C + 3]
        acc = None
        for d in range(3):
            col = posb[:, d:d + 1]
            row = post_ref[b][d:d + 1, :]
            diff = col - row
            sq = diff * diff
            acc = sq if acc is None else acc + sq
        dcur.append(jnp.sqrt(jnp.maximum(acc, 0.0)))
        tbl.append(jnp.dot(XP, wt_ref[...],
                           preferred_element_type=jnp.float32) + bt_ref[...])
        qpe.append(jnp.dot(posb, pw1t_ref[...],
                           preferred_element_type=jnp.float32) + pb1_ref[...])
        qg.append(jnp.dot(xb, wqg_ref[...],
                          preferred_element_type=jnp.float32) + cqg_ref[...])

    gs = [[] for _ in range(_B)]
    vs = [[] for _ in range(_B)]
    for _ in range(_K):
        for b in range(_B):
            mv = jnp.min(dcur[b], axis=1, keepdims=True)
            cand = jnp.where(dcur[b] == mv, iota, big)
            sel = jnp.min(cand, axis=1, keepdims=True)
            hit = iota == sel
            onehot = jnp.where(hit, jnp.float32(1.0), jnp.float32(0.0))
            dcur[b] = jnp.where(hit, inf, dcur[b])

            g = jnp.dot(onehot, tbl[b],
                        preferred_element_type=jnp.float32)
            vj = g[:, 0:_C]
            pej = g[:, _C:2 * _C]
            gkg = g[:, 2 * _C:3 * _C]

            rp = jax.nn.relu(qpe[b] - pej)
            pre1 = (qg[b] - gkg) + jnp.dot(rp, wpg_ref[...],
                                           preferred_element_type=jnp.float32)
            gj = jnp.dot(jax.nn.relu(pre1), gw2t_ref[...],
                         preferred_element_type=jnp.float32)
            ws = vj + jnp.dot(rp, pw2t_ref[...],
                              preferred_element_type=jnp.float32)
            gs[b].append(gj)
            vs[b].append(ws)

    for b in range(_B):
        m = gs[b][0]
        for g in gs[b][1:]:
            m = jnp.maximum(m, g)
        accv = None
        den = None
        for g, w in zip(gs[b], vs[b]):
            e = jnp.exp(g - m)
            accv = e * w if accv is None else accv + e * w
            den = e if den is None else den + e
        out_ref[b] = accv / den + pb2_ref[...]


@jax.jit
def kernel(x, position, Wq, bq, Wk, bk, Wv, bv, pW1, pb1, pW2, pb2,
           gW1, gb1, gW2, gb2):
    s = jnp.float32(1.0 / 8.0)
    xp = jnp.concatenate([x, position], axis=-1)
    post = position.transpose(0, 2, 1)
    z = jnp.zeros((3, _C), jnp.float32)
    zc = jnp.zeros((_C, _C), jnp.float32)
    wkg = s * (gW1 @ Wk).T
    wt = jnp.concatenate([
        jnp.concatenate([Wv.T, zc, wkg], axis=1),
        jnp.concatenate([z, pW1.T, z], axis=1)], axis=0)
    bkg = s * (bk @ gW1.T)
    bt = jnp.concatenate([bv, jnp.zeros((_C,), jnp.float32),
                          bkg]).reshape(1, 3 * _C)
    wqg = s * (gW1 @ Wq).T
    cqg = (s * (bq @ gW1.T) + s * (pb2 @ gW1.T) + gb1).reshape(1, _C)
    wpg = s * (gW1 @ pW2).T

    return pl.pallas_call(
        _ptl_kernel,
        out_shape=jax.ShapeDtypeStruct((_B, _N, _C), jnp.float32),
    )(xp, post, wt, bt, wqg, cqg, pW1.T, pb1.reshape(1, _C), wpg, gW2.T,
      pW2.T, pb2.reshape(1, _C))


# speedup vs baseline: 1.0267x; 1.0267x over previous
import jax
import jax.numpy as jnp
from jax.experimental import pallas as pl
from jax.experimental.pallas import tpu as pltpu

_B, _N, _C, _K = 2, 1024, 64, 16
_R = 1024


def _ptl_kernel(x_ref, pos_ref, post_ref, wqt_ref, bq_ref, wkv_ref, bkv_ref,
                pw1t_ref, pb1_ref, pw2t_ref, pb2_ref,
                gw1t_ref, gb1_ref, gw2t_ref, gb2_ref, out_ref):
    i = pl.program_id(1)
    n0 = i * _R
    X = x_ref[0]
    POS = pos_ref[0]
    posblk = pos_ref[0, pl.ds(n0, _R), :]
    xblk = x_ref[0, pl.ds(n0, _R), :]

    acc = None
    for d in range(3):
        col = posblk[:, d:d + 1]
        row = post_ref[0][d:d + 1, :]
        diff = col - row
        sq = diff * diff
        acc = sq if acc is None else acc + sq
    dist = jnp.sqrt(jnp.maximum(acc, 0.0))

    q = jnp.dot(xblk, wqt_ref[...],
                preferred_element_type=jnp.float32) + bq_ref[...]
    qpe = jnp.dot(posblk, pw1t_ref[...],
                  preferred_element_type=jnp.float32) + pb1_ref[...]
    tbl = jnp.concatenate(
        [jnp.dot(X, wkv_ref[...], preferred_element_type=jnp.float32)
         + bkv_ref[...],
         jnp.dot(POS, pw1t_ref[...], preferred_element_type=jnp.float32)],
        axis=1)

    iota = jax.lax.broadcasted_iota(jnp.int32, (_R, _N), 1).astype(jnp.float32)
    big = jnp.float32(_N)
    inf = jnp.float32(jnp.inf)
    scale = jnp.float32(1.0 / 8.0)

    gs = []
    ws = []
    dcur = dist
    for _ in range(_K):
        mv = jnp.min(dcur, axis=1, keepdims=True)
        cand = jnp.where(dcur == mv, iota, big)
        sel = jnp.min(cand, axis=1, keepdims=True)
        hit = iota == sel
        onehot = jnp.where(hit, jnp.float32(1.0), jnp.float32(0.0))
        dcur = jnp.where(hit, inf, dcur)

        g = jnp.dot(onehot, tbl, preferred_element_type=jnp.float32)
        kj = g[:, 0:_C]
        vj = g[:, _C:2 * _C]
        pej = g[:, 2 * _C:3 * _C]

        pe = jnp.dot(jax.nn.relu(qpe - pej), pw2t_ref[...],
                     preferred_element_type=jnp.float32) + pb2_ref[...]

        aj = (q - kj + pe) * scale
        gj = jnp.dot(jax.nn.relu(
            jnp.dot(aj, gw1t_ref[...], preferred_element_type=jnp.float32)
            + gb1_ref[...]), gw2t_ref[...],
            preferred_element_type=jnp.float32) + gb2_ref[...]
        gs.append(gj)
        ws.append(vj + pe)

    m = gs[0]
    for g in gs[1:]:
        m = jnp.maximum(m, g)
    num = None
    den = None
    for g, w in zip(gs, ws):
        e = jnp.exp(g - m)
        num = e * w if num is None else num + e * w
        den = e if den is None else den + e
    out_ref[0] = num / den


@jax.jit
def kernel(x, position, Wq, bq, Wk, bk, Wv, bv, pW1, pb1, pW2, pb2,
           gW1, gb1, gW2, gb2):
    post = position.transpose(0, 2, 1)
    wkv = jnp.concatenate([Wk.T, Wv.T], axis=1)
    bkv = jnp.concatenate([bk, bv]).reshape(1, 2 * _C)
    w_spec = pl.BlockSpec((_C, _C), lambda b, i: (0, 0))
    b_spec = pl.BlockSpec((1, _C), lambda b, i: (0, 0))
    grid = (_B, _N // _R)
    return pl.pallas_call(
        _ptl_kernel,
        grid=grid,
        in_specs=[
            pl.BlockSpec((1, _N, _C), lambda b, i: (b, 0, 0)),
            pl.BlockSpec((1, _N, 3), lambda b, i: (b, 0, 0)),
            pl.BlockSpec((1, 3, _N), lambda b, i: (b, 0, 0)),
            w_spec, b_spec,
            pl.BlockSpec((_C, 2 * _C), lambda b, i: (0, 0)),
            pl.BlockSpec((1, 2 * _C), lambda b, i: (0, 0)),
            pl.BlockSpec((3, _C), lambda b, i: (0, 0)), b_spec,
            w_spec, b_spec, w_spec, b_spec, w_spec, b_spec,
        ],
        out_specs=pl.BlockSpec((1, _R, _C), lambda b, i: (b, i, 0)),
        out_shape=jax.ShapeDtypeStruct((_B, _N, _C), jnp.float32),
        compiler_params=pltpu.CompilerParams(
            dimension_semantics=("parallel", "arbitrary")),
    )(x, position, post, Wq.T, bq.reshape(1, _C), wkv, bkv,
      pW1.T, pb1.reshape(1, _C), pW2.T, pb2.reshape(1, _C),
      gW1.T, gb1.reshape(1, _C), gW2.T, gb2.reshape(1, _C))


# speedup vs baseline: 1.0413x; 1.0141x over previous
import jax
import jax.numpy as jnp
from jax.experimental import pallas as pl
from jax.experimental.pallas import tpu as pltpu

_B, _N, _C, _K = 2, 1024, 64, 16
_R = 1024


def _ptl_kernel(x_ref, pos_ref, post_ref, wqt_ref, bq_ref, wkv_ref, bkv_ref,
                pw1t_ref, pb1_ref, pw2t_ref, pb2_ref,
                gw1t_ref, gb1_ref, gw2t_ref, gb2_ref, out_ref):
    i = pl.program_id(1)
    n0 = i * _R
    X = x_ref[0]
    POS = pos_ref[0]
    posblk = pos_ref[0, pl.ds(n0, _R), :]
    xblk = x_ref[0, pl.ds(n0, _R), :]

    acc = None
    for d in range(3):
        col = posblk[:, d:d + 1]
        row = post_ref[0][d:d + 1, :]
        diff = col - row
        sq = diff * diff
        acc = sq if acc is None else acc + sq
    dist = jnp.sqrt(acc)

    q = jnp.dot(xblk, wqt_ref[...],
                preferred_element_type=jnp.float32) + bq_ref[...]
    qpe = jnp.dot(posblk, pw1t_ref[...],
                  preferred_element_type=jnp.float32) + pb1_ref[...]
    tbl = jnp.concatenate(
        [jnp.dot(X, wkv_ref[...], preferred_element_type=jnp.float32)
         + bkv_ref[...],
         jnp.dot(POS, pw1t_ref[...], preferred_element_type=jnp.float32)],
        axis=1)

    iota = jax.lax.broadcasted_iota(jnp.int32, (_R, _N), 1).astype(jnp.float32)
    big = jnp.float32(_N)
    inf = jnp.float32(jnp.inf)

    gs = []
    ws = []
    dcur = dist
    for _ in range(_K):
        mv = jnp.min(dcur, axis=1, keepdims=True)
        cand = jnp.where(dcur == mv, iota, big)
        sel = jnp.min(cand, axis=1, keepdims=True)
        hit = iota == sel
        onehot = jnp.where(hit, jnp.float32(1.0), jnp.float32(0.0))
        dcur = jnp.where(hit, inf, dcur)

        g = jnp.dot(onehot, tbl, preferred_element_type=jnp.float32)
        kj = g[:, 0:_C]
        vj = g[:, _C:2 * _C]
        pej = g[:, 2 * _C:3 * _C]

        pe = jnp.dot(jax.nn.relu(qpe - pej), pw2t_ref[...],
                     preferred_element_type=jnp.float32) + pb2_ref[...]

        aj = q - kj + pe
        gj = jnp.dot(jax.nn.relu(
            jnp.dot(aj, gw1t_ref[...], preferred_element_type=jnp.float32)
            + gb1_ref[...]), gw2t_ref[...],
            preferred_element_type=jnp.float32) + gb2_ref[...]
        gs.append(gj)
        ws.append(vj + pe)

    m = gs[0]
    for g in gs[1:]:
        m = jnp.maximum(m, g)
    num = None
    den = None
    for g, w in zip(gs, ws):
        e = jnp.exp(g - m)
        num = e * w if num is None else num + e * w
        den = e if den is None else den + e
    out_ref[0] = num / den


@jax.jit
def kernel(x, position, Wq, bq, Wk, bk, Wv, bv, pW1, pb1, pW2, pb2,
           gW1, gb1, gW2, gb2):
    post = position.transpose(0, 2, 1)
    wkv = jnp.concatenate([Wk.T, Wv.T], axis=1)
    bkv = jnp.concatenate([bk, bv]).reshape(1, 2 * _C)
    w_spec = pl.BlockSpec((_C, _C), lambda b, i: (0, 0))
    b_spec = pl.BlockSpec((1, _C), lambda b, i: (0, 0))
    grid = (_B, _N // _R)
    return pl.pallas_call(
        _ptl_kernel,
        grid=grid,
        in_specs=[
            pl.BlockSpec((1, _N, _C), lambda b, i: (b, 0, 0)),
            pl.BlockSpec((1, _N, 3), lambda b, i: (b, 0, 0)),
            pl.BlockSpec((1, 3, _N), lambda b, i: (b, 0, 0)),
            w_spec, b_spec,
            pl.BlockSpec((_C, 2 * _C), lambda b, i: (0, 0)),
            pl.BlockSpec((1, 2 * _C), lambda b, i: (0, 0)),
            pl.BlockSpec((3, _C), lambda b, i: (0, 0)), b_spec,
            w_spec, b_spec, w_spec, b_spec, w_spec, b_spec,
        ],
        out_specs=pl.BlockSpec((1, _R, _C), lambda b, i: (b, i, 0)),
        out_shape=jax.ShapeDtypeStruct((_B, _N, _C), jnp.float32),
        compiler_params=pltpu.CompilerParams(
            dimension_semantics=("parallel", "arbitrary")),
    )(x, position, post, Wq.T, bq.reshape(1, _C), wkv, bkv,
      pW1.T, pb1.reshape(1, _C), pW2.T, pb2.reshape(1, _C),
      gW1.T * jnp.float32(1.0 / 8.0), gb1.reshape(1, _C), gW2.T,
      gb2.reshape(1, _C))
